# Initial kernel scaffold; baseline (speedup 1.0000x reference)
import jax
import jax.numpy as jnp
from jax.experimental import pallas as pl


def kernel(indices, table, W, b):
    raise NotImplementedError("write your pallas kernel here")


# speedup vs baseline: 94.0940x; 94.0940x over previous
import functools

import jax
import jax.numpy as jnp
from jax import lax
from jax.experimental import pallas as pl
from jax.experimental.pallas import tpu as pltpu
from jax.experimental.pallas import tpu_sc as plsc

N_VOCAB = 8
DIM = 10
LANES = 16
NUM_CORES = 2
NUM_SUBCORES = 16
NUM_WORKERS = NUM_CORES * NUM_SUBCORES


def _sc_lut_remap(total, sub):
    per_w = total // NUM_WORKERS
    nsub = per_w // sub
    n_vec = sub // LANES

    mesh = plsc.VectorSubcoreMesh(core_axis_name="c", subcore_axis_name="s")

    @functools.partial(
        pl.kernel,
        out_type=jax.ShapeDtypeStruct((total,), jnp.float32),
        mesh=mesh,
        scratch_types=[
            pltpu.VMEM((LANES, LANES), jnp.float32),
            pltpu.VMEM((LANES, LANES), jnp.float32),
            pltpu.VMEM((sub,), jnp.int32),
            pltpu.VMEM((sub,), jnp.float32),
        ],
    )
    def k(idx_hbm, tab_hbm, wb_hbm, out_hbm, tab_v, wb_v, idx_v, out_v):
        wid = lax.axis_index("s") * NUM_CORES + lax.axis_index("c")
        base = wid * per_w

        pltpu.sync_copy(tab_hbm, tab_v)
        pltpu.sync_copy(wb_hbm, wb_v)

        acc = jnp.zeros((LANES,), jnp.float32)
        for d in range(DIM):
            acc = acc + tab_v[d] * wb_v[d]
        z = acc + wb_v[DIM]
        s = 1.0 / (1.0 + jnp.exp(-z))

        def sub_body(j, carry):
            off = base + j * sub
            pltpu.sync_copy(idx_hbm.at[pl.ds(off, sub)], idx_v)

            def inner(i, c):
                v = idx_v[pl.ds(i * LANES, LANES)]
                out_v[pl.ds(i * LANES, LANES)] = lax.gather(
                    s, v[:, None],
                    dimension_numbers=lax.GatherDimensionNumbers(
                        offset_dims=(), collapsed_slice_dims=(0,),
                        start_index_map=(0,)),
                    slice_sizes=(1,),
                    mode=lax.GatherScatterMode.PROMISE_IN_BOUNDS)
                return c

            lax.fori_loop(0, n_vec, inner, 0, unroll=4)
            pltpu.sync_copy(out_v, out_hbm.at[pl.ds(off, sub)])
            return carry

        lax.fori_loop(0, nsub, sub_body, 0)

    return k


def kernel(indices, table, W, b):
    bsz, seq = indices.shape
    total = bsz * seq
    sub = 25600

    idx_flat = indices.reshape(total).astype(jnp.int32)
    tab_t = jnp.zeros((LANES, LANES), jnp.float32).at[:DIM, :N_VOCAB].set(table.T)
    wb = (
        jnp.zeros((LANES, LANES), jnp.float32)
        .at[:DIM, :].set(jnp.broadcast_to(W[0][:, None], (DIM, LANES)))
        .at[DIM, :].set(b[0])
    )

    out_flat = _sc_lut_remap(total, sub)(idx_flat, tab_t, wb)
    return out_flat.reshape(bsz, seq, 1)


# speedup vs baseline: 98.7712x; 1.0497x over previous
import functools

import jax
import jax.numpy as jnp
from jax import lax
from jax.experimental import pallas as pl
from jax.experimental.pallas import tpu as pltpu
from jax.experimental.pallas import tpu_sc as plsc

N_VOCAB = 8
DIM = 10
LANES = 16
NUM_CORES = 2
NUM_SUBCORES = 16
NUM_WORKERS = NUM_CORES * NUM_SUBCORES


def _sc_lut_remap(total, sub):
    per_w = total // NUM_WORKERS
    nsub = per_w // sub
    n_vec = sub // LANES

    mesh = plsc.VectorSubcoreMesh(core_axis_name="c", subcore_axis_name="s")

    @functools.partial(
        pl.kernel,
        out_type=jax.ShapeDtypeStruct((total,), jnp.float32),
        mesh=mesh,
        scratch_types=[
            pltpu.VMEM((LANES, LANES), jnp.float32),
            pltpu.VMEM((LANES, LANES), jnp.float32),
            pltpu.VMEM((sub,), jnp.int32),
            pltpu.VMEM((sub,), jnp.int32),
            pltpu.VMEM((sub,), jnp.float32),
            pltpu.VMEM((sub,), jnp.float32),
            pltpu.SemaphoreType.DMA,
            pltpu.SemaphoreType.DMA,
            pltpu.SemaphoreType.DMA,
            pltpu.SemaphoreType.DMA,
        ],
    )
    def k(idx_hbm, tab_hbm, wb_hbm, out_hbm, tab_v, wb_v,
          idx0, idx1, out0, out1, si0, si1, so0, so1):
        wid = lax.axis_index("s") * NUM_CORES + lax.axis_index("c")
        base = wid * per_w
        idx_bufs, out_bufs = (idx0, idx1), (out0, out1)
        sin, sout = (si0, si1), (so0, so1)

        pltpu.sync_copy(tab_hbm, tab_v)
        pltpu.sync_copy(wb_hbm, wb_v)

        acc = jnp.zeros((LANES,), jnp.float32)
        for d in range(DIM):
            acc = acc + tab_v[d] * wb_v[d]
        z = acc + wb_v[DIM]
        s = 1.0 / (1.0 + jnp.exp(-z))

        def compute(ib, ob):
            def inner(i, c):
                v = ib[pl.ds(i * LANES, LANES)]
                ob[pl.ds(i * LANES, LANES)] = lax.gather(
                    s, v[:, None],
                    dimension_numbers=lax.GatherDimensionNumbers(
                        offset_dims=(), collapsed_slice_dims=(0,),
                        start_index_map=(0,)),
                    slice_sizes=(1,),
                    mode=lax.GatherScatterMode.PROMISE_IN_BOUNDS)
                return c

            lax.fori_loop(0, n_vec, inner, 0, unroll=8)

        h_in = [None] * nsub
        h_out = [None] * nsub
        h_in[0] = pltpu.async_copy(idx_hbm.at[pl.ds(base, sub)], idx0, si0)
        h_in[1] = pltpu.async_copy(idx_hbm.at[pl.ds(base + sub, sub)], idx1, si1)
        for j in range(nsub):
            slot = j % 2
            h_in[j].wait()
            if j >= 2:
                h_out[j - 2].wait()
            compute(idx_bufs[slot], out_bufs[slot])
            h_out[j] = pltpu.async_copy(
                out_bufs[slot], out_hbm.at[pl.ds(base + j * sub, sub)],
                sout[slot])
            if j + 2 < nsub:
                h_in[j + 2] = pltpu.async_copy(
                    idx_hbm.at[pl.ds(base + (j + 2) * sub, sub)],
                    idx_bufs[slot], sin[slot])
        h_out[nsub - 2].wait()
        h_out[nsub - 1].wait()

    return k


def kernel(indices, table, W, b):
    bsz, seq = indices.shape
    total = bsz * seq
    sub = 25600

    idx_flat = indices.reshape(total).astype(jnp.int32)
    tab_t = jnp.zeros((LANES, LANES), jnp.float32).at[:DIM, :N_VOCAB].set(table.T)
    wb = (
        jnp.zeros((LANES, LANES), jnp.float32)
        .at[:DIM, :].set(jnp.broadcast_to(W[0][:, None], (DIM, LANES)))
        .at[DIM, :].set(b[0])
    )

    out_flat = _sc_lut_remap(total, sub)(idx_flat, tab_t, wb)
    return out_flat.reshape(bsz, seq, 1)
